# Initial kernel scaffold; baseline (speedup 1.0000x reference)
import jax
import jax.numpy as jnp
from jax.experimental import pallas as pl


def kernel(x, edge_index, W, gamma, beta):
    raise NotImplementedError("write your pallas kernel here")


# speedup vs baseline: 3.4202x; 3.4202x over previous
import functools

import jax
import jax.numpy as jnp
from jax import lax
from jax.experimental import pallas as pl
from jax.experimental.pallas import tpu as pltpu
from jax.experimental.pallas import tpu_sc as plsc

EPS = 1e-5
NC = 2
NS = 16
NW = NC * NS
LANES = 16
B = 128


def _mm_body(x_ref, w_ref, y1_ref, y2_ref):
    xb = x_ref[...]
    w = w_ref[...]
    c = w.shape[1] // 2
    w2 = w[:, c:]
    d = w[:, :c] - w2
    dn = (((1,), (1,)), ((), ()))
    y1_ref[...] = lax.dot_general(xb, d, dn, preferred_element_type=jnp.float32)
    y2_ref[...] = lax.dot_general(xb, w2, dn, preferred_element_type=jnp.float32)


def _node_matmuls(x, W):
    n, c = x.shape
    bn = 1000
    grid = (n // bn,)
    return pl.pallas_call(
        _mm_body,
        grid=grid,
        in_specs=[
            pl.BlockSpec((bn, c), lambda i: (i, 0)),
            pl.BlockSpec((c, 2 * c), lambda i: (0, 0)),
        ],
        out_specs=[
            pl.BlockSpec((bn, c), lambda i: (i, 0)),
            pl.BlockSpec((bn, c), lambda i: (i, 0)),
        ],
        out_shape=[
            jax.ShapeDtypeStruct((n, c), jnp.float32),
            jax.ShapeDtypeStruct((n, c), jnp.float32),
        ],
    )(x, W)


def _pass1_body(nblk, y1_hbm, y2_hbm, row_hbm, col_hbm, h_hbm, stats_hbm,
                ridx_v, cidx_v, g1_v, g2_v, hbuf_v, sacc_v, sem1, sem2):
    cid = lax.axis_index("c")
    sid = lax.axis_index("s")
    wid = sid * NC + cid
    max_i = (nblk + NW - 1) // NW

    zero = jnp.zeros((LANES,), jnp.float32)
    for c in range(8):
        sacc_v[0, pl.ds(c * LANES, LANES)] = zero
        sacc_v[1, pl.ds(c * LANES, LANES)] = zero

    def blk_body(i, _):
        blk = wid + NW * i

        @pl.when(blk < nblk)
        def _():
            base = blk * B
            pltpu.sync_copy(row_hbm.at[pl.ds(base, B)], ridx_v)
            pltpu.sync_copy(col_hbm.at[pl.ds(base, B)], cidx_v)
            cp1 = pltpu.async_copy(y1_hbm.at[ridx_v], g1_v, sem1)
            cp2 = pltpu.async_copy(y2_hbm.at[cidx_v], g2_v, sem2)
            cp1.wait()
            cp2.wait()

            def e_body(e, sq):
                s = list(sq[:8])
                q = list(sq[8:])
                for c in range(8):
                    v1 = g1_v[e, pl.ds(c * LANES, LANES)]
                    v2 = g2_v[e, pl.ds(c * LANES, LANES)]
                    h = v1 + v2
                    hbuf_v[e, pl.ds(c * LANES, LANES)] = h
                    s[c] = s[c] + h
                    q[c] = q[c] + h * h
                return tuple(s) + tuple(q)

            carry = lax.fori_loop(0, B, e_body, (zero,) * 16)
            for c in range(8):
                plsc.addupdate(sacc_v.at[0, pl.ds(c * LANES, LANES)], carry[c])
                plsc.addupdate(
                    sacc_v.at[1, pl.ds(c * LANES, LANES)], carry[8 + c])
            pltpu.sync_copy(hbuf_v, h_hbm.at[pl.ds(base, B)])

        return 0

    lax.fori_loop(0, max_i, blk_body, 0)
    pltpu.sync_copy(sacc_v, stats_hbm.at[wid])


def _pass1(y1, y2, row, col):
    n, c = y1.shape
    e = row.shape[0]
    nblk = e // B
    mesh = plsc.VectorSubcoreMesh(
        core_axis_name="c", subcore_axis_name="s", num_cores=NC,
        num_subcores=NS)
    return pl.kernel(
        functools.partial(_pass1_body, nblk),
        out_type=(
            jax.ShapeDtypeStruct((e, c), jnp.float32),
            jax.ShapeDtypeStruct((NW, 2, c), jnp.float32),
        ),
        mesh=mesh,
        scratch_types=[
            pltpu.VMEM((B,), jnp.int32),
            pltpu.VMEM((B,), jnp.int32),
            pltpu.VMEM((B, c), jnp.float32),
            pltpu.VMEM((B, c), jnp.float32),
            pltpu.VMEM((B, c), jnp.float32),
            pltpu.VMEM((2, c), jnp.float32),
            pltpu.SemaphoreType.DMA,
            pltpu.SemaphoreType.DMA,
        ],
    )(y1, y2, row, col)


def _rsqrt16(xv):
    i = lax.bitcast_convert_type(xv, jnp.int32)
    i = jnp.full((LANES,), 0x5F3759DF, jnp.int32) - lax.shift_right_arithmetic(
        i, jnp.full((LANES,), 1, jnp.int32))
    y = lax.bitcast_convert_type(i, jnp.float32)
    for _ in range(3):
        y = y * (1.5 - 0.5 * xv * y * y)
    return y


def _pass2_body(nblk, n_edges, h_hbm, row_hbm, stats_hbm, gamma_hbm, beta_hbm,
                part_hbm, stats_v, g_v, b_v, ridx_v, hbuf_v, acc_sh, sem):
    cid = lax.axis_index("c")
    sid = lax.axis_index("s")
    wid = sid * NC + cid
    max_i = (nblk + NW - 1) // NW
    npad = acc_sh.shape[0]
    rows_per = npad // NS

    pltpu.sync_copy(stats_hbm, stats_v)
    pltpu.sync_copy(gamma_hbm, g_v)
    pltpu.sync_copy(beta_hbm, b_v)

    zero = jnp.zeros((LANES,), jnp.float32)
    scale = []
    shift = []
    inv_e = 1.0 / float(n_edges)
    for c in range(8):
        def w_body(w, sq, c=c):
            s, q = sq
            return (s + stats_v[w, 0, pl.ds(c * LANES, LANES)],
                    q + stats_v[w, 1, pl.ds(c * LANES, LANES)])

        s, q = lax.fori_loop(0, NW, w_body, (zero, zero))
        mean = s * inv_e
        var = q * inv_e - mean * mean
        inv_std = _rsqrt16(var + EPS)
        a = g_v[pl.ds(c * LANES, LANES)] * inv_std
        scale.append(a)
        shift.append(b_v[pl.ds(c * LANES, LANES)] - mean * a)

    def z_body(e, _):
        for c in range(8):
            hbuf_v[e, pl.ds(c * LANES, LANES)] = zero
        return 0

    lax.fori_loop(0, B, z_body, 0)
    base_row = sid * rows_per
    for k in range(rows_per // B):
        pltpu.sync_copy(hbuf_v, acc_sh.at[pl.ds(base_row + k * B, B)])
    plsc.subcore_barrier()

    def blk_body(i, _):
        blk = wid + NW * i

        @pl.when(blk < nblk)
        def _():
            base = blk * B
            pltpu.sync_copy(row_hbm.at[pl.ds(base, B)], ridx_v)
            pltpu.sync_copy(h_hbm.at[pl.ds(base, B)], hbuf_v)

            def e_body(e, _):
                for c in range(8):
                    h = hbuf_v[e, pl.ds(c * LANES, LANES)]
                    t = scale[c] * h + shift[c]
                    sg = 1.0 / (1.0 + jnp.exp(-t))
                    hbuf_v[e, pl.ds(c * LANES, LANES)] = t * sg
                return 0

            lax.fori_loop(0, B, e_body, 0)
            pltpu.sync_copy(hbuf_v, acc_sh.at[ridx_v], add=True)

        return 0

    lax.fori_loop(0, max_i, blk_body, 0)
    plsc.subcore_barrier()
    pltpu.sync_copy(acc_sh.at[pl.ds(base_row, rows_per)],
                    part_hbm.at[cid, pl.ds(base_row, rows_per)])


def _pass2(h, row, stats, gamma, beta, n):
    e, c = h.shape
    nblk = e // B
    npad = ((n + NS * B - 1) // (NS * B)) * NS * B
    mesh = plsc.VectorSubcoreMesh(
        core_axis_name="c", subcore_axis_name="s", num_cores=NC,
        num_subcores=NS)
    return pl.kernel(
        functools.partial(_pass2_body, nblk, e),
        out_type=jax.ShapeDtypeStruct((NC, npad, c), jnp.float32),
        mesh=mesh,
        scratch_types=[
            pltpu.VMEM((NW, 2, c), jnp.float32),
            pltpu.VMEM((c,), jnp.float32),
            pltpu.VMEM((c,), jnp.float32),
            pltpu.VMEM((B,), jnp.int32),
            pltpu.VMEM((B, c), jnp.float32),
            pltpu.VMEM_SHARED((npad, c), jnp.float32),
            pltpu.SemaphoreType.DMA,
        ],
    )(h, row, stats, gamma, beta)


def _add_body(p_ref, o_ref):
    o_ref[...] = p_ref[0] + p_ref[1]


def _combine(part, n):
    c = part.shape[2]
    bn = 1000
    return pl.pallas_call(
        _add_body,
        grid=(n // bn,),
        in_specs=[pl.BlockSpec((2, bn, c), lambda i: (0, i, 0))],
        out_specs=pl.BlockSpec((bn, c), lambda i: (i, 0)),
        out_shape=jax.ShapeDtypeStruct((n, c), jnp.float32),
    )(part)


def kernel(x, edge_index, W, gamma, beta):
    n = x.shape[0]
    row = edge_index[0]
    col = edge_index[1]
    y1, y2 = _node_matmuls(x, W)
    h, stats = _pass1(y1, y2, row, col)
    part = _pass2(h, row, stats, gamma, beta, n)
    return _combine(part, n)


# speedup vs baseline: 3.8581x; 1.1280x over previous
import functools

import jax
import jax.numpy as jnp
from jax import lax
from jax.experimental import pallas as pl
from jax.experimental.pallas import tpu as pltpu
from jax.experimental.pallas import tpu_sc as plsc

EPS = 1e-5
NC = 2
NS = 16
NW = NC * NS
LANES = 16
B = 128


def _mm_body(x_ref, w_ref, y1_ref, y2_ref):
    xb = x_ref[...]
    w = w_ref[...]
    c = w.shape[1] // 2
    w2 = w[:, c:]
    d = w[:, :c] - w2
    dn = (((1,), (1,)), ((), ()))
    y1_ref[...] = lax.dot_general(xb, d, dn, preferred_element_type=jnp.float32)
    y2_ref[...] = lax.dot_general(xb, w2, dn, preferred_element_type=jnp.float32)


def _node_matmuls(x, W):
    n, c = x.shape
    bn = n // 8
    grid = (n // bn,)
    return pl.pallas_call(
        _mm_body,
        grid=grid,
        in_specs=[
            pl.BlockSpec((bn, c), lambda i: (i, 0)),
            pl.BlockSpec((c, 2 * c), lambda i: (0, 0)),
        ],
        out_specs=[
            pl.BlockSpec((bn, c), lambda i: (i, 0)),
            pl.BlockSpec((bn, c), lambda i: (i, 0)),
        ],
        out_shape=[
            jax.ShapeDtypeStruct((n, c), jnp.float32),
            jax.ShapeDtypeStruct((n, c), jnp.float32),
        ],
    )(x, W)


def _pass1_body(nb, y1_hbm, y2_hbm, row_hbm, col_hbm, h_hbm, stats_hbm,
                ridx_v, cidx_v, ri0, ri1, ci0, ci1, g1a, g1b, g2a, g2b,
                sacc_v, g1s0, g1s1, g2s0, g2s1, hw0, hw1):
    cid = lax.axis_index("c")
    sid = lax.axis_index("s")
    wid = sid * NC + cid
    g1buf = (g1a, g1b)
    g2buf = (g2a, g2b)
    g1sem = (g1s0, g1s1)
    g2sem = (g2s0, g2s1)
    hwsem = (hw0, hw1)
    blk0 = pl.multiple_of(wid * nb, 8)
    ebase0 = pl.multiple_of(blk0 * B, B)

    pltpu.sync_copy(row_hbm.at[pl.ds(blk0, nb)], ridx_v)
    pltpu.sync_copy(col_hbm.at[pl.ds(blk0, nb)], cidx_v)

    zero = jnp.zeros((LANES,), jnp.float32)
    for c in range(8):
        sacc_v[0, pl.ds(c * LANES, LANES)] = zero
        sacc_v[1, pl.ds(c * LANES, LANES)] = zero

    def compute_block(gbuf1, gbuf2):
        def e_body(e, sq):
            s = list(sq[:8])
            q = list(sq[8:])
            for c in range(8):
                v1 = gbuf1[e, pl.ds(c * LANES, LANES)]
                v2 = gbuf2[e, pl.ds(c * LANES, LANES)]
                h = v1 + v2
                gbuf1[e, pl.ds(c * LANES, LANES)] = h
                s[c] = s[c] + h
                q[c] = q[c] + h * h
            return tuple(s) + tuple(q)

        carry = lax.fori_loop(0, B, e_body, (zero,) * 16)
        for c in range(8):
            plsc.addupdate(sacc_v.at[0, pl.ds(c * LANES, LANES)], carry[c])
            plsc.addupdate(sacc_v.at[1, pl.ds(c * LANES, LANES)], carry[8 + c])

    def blk_one(b, _):
        for k in range(B // LANES):
            ri0[pl.ds(k * LANES, LANES)] = ridx_v[b, 0, pl.ds(k * LANES, LANES)]
            ci0[pl.ds(k * LANES, LANES)] = cidx_v[b, 0, pl.ds(k * LANES, LANES)]
        dA1 = pltpu.async_copy(y1_hbm.at[ri0], g1a, g1s0)
        dA2 = pltpu.async_copy(y2_hbm.at[ci0], g2a, g2s0)
        dA1.wait()
        dA2.wait()
        compute_block(g1a, g2a)
        pltpu.sync_copy(g1a, h_hbm.at[pl.ds(ebase0 + b * B, B)])
        return 0

    lax.fori_loop(0, nb, blk_one, 0)
    pltpu.sync_copy(sacc_v, stats_hbm.at[wid])


def _pass1(y1, y2, row2d, col2d):
    n, c = y1.shape
    nblk = row2d.shape[0]
    e = nblk * B
    nb = nblk // NW
    mesh = plsc.VectorSubcoreMesh(
        core_axis_name="c", subcore_axis_name="s", num_cores=NC,
        num_subcores=NS)
    return pl.kernel(
        functools.partial(_pass1_body, nb),
        out_type=(
            jax.ShapeDtypeStruct((e, c), jnp.float32),
            jax.ShapeDtypeStruct((NW, 2, c), jnp.float32),
        ),
        mesh=mesh,
        scratch_types=[
            pltpu.VMEM((nb, 1, B), jnp.int32),
            pltpu.VMEM((nb, 1, B), jnp.int32),
            pltpu.VMEM((B,), jnp.int32),
            pltpu.VMEM((B,), jnp.int32),
            pltpu.VMEM((B,), jnp.int32),
            pltpu.VMEM((B,), jnp.int32),
            pltpu.VMEM((B, c), jnp.float32),
            pltpu.VMEM((B, c), jnp.float32),
            pltpu.VMEM((B, c), jnp.float32),
            pltpu.VMEM((B, c), jnp.float32),
            pltpu.VMEM((2, c), jnp.float32),
            pltpu.SemaphoreType.DMA,
            pltpu.SemaphoreType.DMA,
            pltpu.SemaphoreType.DMA,
            pltpu.SemaphoreType.DMA,
            pltpu.SemaphoreType.DMA,
            pltpu.SemaphoreType.DMA,
        ],
    )(y1, y2, row2d, col2d)


def _rsqrt16(xv):
    i = lax.bitcast_convert_type(xv, jnp.int32)
    i = jnp.full((LANES,), 0x5F3759DF, jnp.int32) - lax.shift_right_arithmetic(
        i, jnp.full((LANES,), 1, jnp.int32))
    y = lax.bitcast_convert_type(i, jnp.float32)
    for _ in range(3):
        y = y * (1.5 - 0.5 * xv * y * y)
    return y


def _pass2_body(nb, n_edges, h_hbm, row_hbm, stats_hbm, gamma_hbm, beta_hbm,
                part_hbm, stats_v, sq_v, g_v, b_v, ridx_v, si0, si1, hb0,
                hb1, acc_sh, hr0, hr1, sc0, sc1):
    cid = lax.axis_index("c")
    sid = lax.axis_index("s")
    wid = sid * NC + cid
    npad = acc_sh.shape[0]
    rows_per = npad // NS
    hbuf = (hb0, hb1)
    hrsem = (hr0, hr1)
    scsem = (sc0, sc1)
    blk0 = pl.multiple_of(wid * nb, 8)
    ebase0 = pl.multiple_of(blk0 * B, B)
    schunk = stats_v.shape[0]

    pltpu.sync_copy(row_hbm.at[pl.ds(blk0, nb)], ridx_v)
    pltpu.sync_copy(gamma_hbm, g_v)
    pltpu.sync_copy(beta_hbm, b_v)

    zero = jnp.zeros((LANES,), jnp.float32)
    inv_e = 1.0 / float(n_edges)
    for c in range(8):
        sq_v[0, pl.ds(c * LANES, LANES)] = zero
        sq_v[1, pl.ds(c * LANES, LANES)] = zero
    for k in range(NW // schunk):
        pltpu.sync_copy(stats_hbm.at[pl.ds(k * schunk, schunk)], stats_v)

        def w_body(w, _):
            for c in range(8):
                plsc.addupdate(sq_v.at[0, pl.ds(c * LANES, LANES)],
                               stats_v[w, 0, pl.ds(c * LANES, LANES)])
                plsc.addupdate(sq_v.at[1, pl.ds(c * LANES, LANES)],
                               stats_v[w, 1, pl.ds(c * LANES, LANES)])
            return 0

        lax.fori_loop(0, schunk, w_body, 0)
    scale = []
    shift = []
    for c in range(8):
        mean = sq_v[0, pl.ds(c * LANES, LANES)] * inv_e
        var = sq_v[1, pl.ds(c * LANES, LANES)] * inv_e - mean * mean
        inv_std = _rsqrt16(var + EPS)
        a = g_v[pl.ds(c * LANES, LANES)] * inv_std
        scale.append(a)
        shift.append(b_v[pl.ds(c * LANES, LANES)] - mean * a)

    def z_body(e, _):
        for c in range(8):
            hb0[e, pl.ds(c * LANES, LANES)] = zero
        return 0

    lax.fori_loop(0, B, z_body, 0)
    base_row = sid * rows_per
    for k in range(rows_per // B):
        pltpu.sync_copy(hb0, acc_sh.at[pl.ds(base_row + k * B, B)])
    rem = rows_per % B
    if rem:
        pltpu.sync_copy(
            hb0.at[pl.ds(0, rem)],
            acc_sh.at[pl.ds(base_row + (rows_per // B) * B, rem)])
    plsc.subcore_barrier()

    def silu_block(buf):
        def e_body(e, _):
            for c in range(8):
                h = buf[e, pl.ds(c * LANES, LANES)]
                t = scale[c] * h + shift[c]
                sg = 1.0 / (1.0 + jnp.exp(-t))
                buf[e, pl.ds(c * LANES, LANES)] = t * sg
            return 0

        lax.fori_loop(0, B, e_body, 0)

    def blk_one(b, _):
        for k in range(B // LANES):
            si0[pl.ds(k * LANES, LANES)] = ridx_v[b, 0, pl.ds(k * LANES, LANES)]
        pltpu.sync_copy(h_hbm.at[pl.ds(ebase0 + b * B, B)], hb0)
        silu_block(hb0)
        pltpu.sync_copy(hb0, acc_sh.at[si0], add=True)
        return 0

    lax.fori_loop(0, nb, blk_one, 0)
    plsc.subcore_barrier()
    pltpu.sync_copy(acc_sh.at[pl.ds(base_row, rows_per)],
                    part_hbm.at[cid, pl.ds(base_row, rows_per)])


def _pass2(h, row2d, stats, gamma, beta, n, n_edges):
    e, c = h.shape
    nblk = row2d.shape[0]
    nb = nblk // NW
    rows_per = (((n + NS - 1) // NS) + 7) // 8 * 8
    npad = NS * rows_per
    mesh = plsc.VectorSubcoreMesh(
        core_axis_name="c", subcore_axis_name="s", num_cores=NC,
        num_subcores=NS)
    return pl.kernel(
        functools.partial(_pass2_body, nb, n_edges),
        out_type=jax.ShapeDtypeStruct((NC, npad, c), jnp.float32),
        mesh=mesh,
        scratch_types=[
            pltpu.VMEM((8, 2, c), jnp.float32),
            pltpu.VMEM((2, c), jnp.float32),
            pltpu.VMEM((c,), jnp.float32),
            pltpu.VMEM((c,), jnp.float32),
            pltpu.VMEM((nb, 1, B), jnp.int32),
            pltpu.VMEM((B,), jnp.int32),
            pltpu.VMEM((B,), jnp.int32),
            pltpu.VMEM((B, c), jnp.float32),
            pltpu.VMEM((B, c), jnp.float32),
            pltpu.VMEM_SHARED((npad, c), jnp.float32),
            pltpu.SemaphoreType.DMA,
            pltpu.SemaphoreType.DMA,
            pltpu.SemaphoreType.DMA,
            pltpu.SemaphoreType.DMA,
        ],
    )(h, row2d, stats, gamma, beta)


def _add_body(p_ref, o_ref):
    o_ref[...] = p_ref[0] + p_ref[1]


def _combine(part, n):
    c = part.shape[2]
    bn = 1000
    return pl.pallas_call(
        _add_body,
        grid=(n // bn,),
        in_specs=[pl.BlockSpec((2, bn, c), lambda i: (0, i, 0))],
        out_specs=pl.BlockSpec((bn, c), lambda i: (i, 0)),
        out_shape=jax.ShapeDtypeStruct((n, c), jnp.float32),
    )(part)


def kernel(x, edge_index, W, gamma, beta):
    n, c = x.shape
    e = edge_index.shape[1]
    row = edge_index[0]
    col = edge_index[1]

    bn = 1256
    n_pad = ((n + bn) // bn) * bn
    nblk = ((e + NW * B - 1) // (NW * B)) * NW
    e_pad = nblk * B
    pad_idx = n + (jnp.arange(e_pad - e, dtype=jnp.int32) % (n_pad - n))
    row2d = jnp.concatenate([row, pad_idx]).reshape(nblk, 1, B)
    col2d = jnp.concatenate([col, pad_idx]).reshape(nblk, 1, B)
    xp = jnp.concatenate([x, jnp.zeros((n_pad - n, c), jnp.float32)])

    y1, y2 = _node_matmuls(xp, W)
    h, stats = _pass1(y1, y2, row2d, col2d)
    part = _pass2(h, row2d, stats, gamma, beta, n, e)
    return _combine(part, n)
